# Initial kernel scaffold; baseline (speedup 1.0000x reference)
import jax
import jax.numpy as jnp
from jax.experimental import pallas as pl


def kernel(own_obs, agent_obs, target_obs, edge_index, edge_attr, phi_a_W, phi_a_b, rho_a_W, rho_a_b, phi_t_W, phi_t_b, rho_t_W, rho_t_b, gcn_in_W, gcn_in_b, gcn_taps_W, gcn_layer_b, gcn_out_W, gcn_out_b):
    raise NotImplementedError("write your pallas kernel here")


# speedup vs baseline: 3.5931x; 3.5931x over previous
import functools

import jax
import jax.numpy as jnp
from jax import lax
from jax.experimental import pallas as pl
from jax.experimental.pallas import tpu as pltpu
from jax.experimental.pallas import tpu_sc as plsc

N_AG = 10000
N_TG = 2048
N_ALL = N_AG + N_TG
NA_P = 10240
NC_P = 12288
NT_P = 2304
E = 160000
NWORK = 32
CH = 128
E_P = 163840
NCH = 32
R2 = 0.02 * 0.02
K_NBR = 10
BIG = 1e30
KNN_BR = 128


def _knn_body(q_ref, p_ref, out_ref, e_ref, *, n_pad, br, row_base):
    i = pl.program_id(0)
    qx = q_ref[:, 0:1]
    qy = q_ref[:, 1:2]
    px = p_ref[0:1, :]
    py = p_ref[1:2, :]
    dx = qx - px
    dy = qy - py
    d2 = dx * dx + dy * dy
    col = lax.broadcasted_iota(jnp.int32, (1, n_pad), 1)
    gi = lax.broadcasted_iota(jnp.int32, (br, 1), 0) + (i * br + row_base)
    is_self = col == gi
    e_ref[...] = jnp.where(is_self | (d2 > R2), BIG, d2)

    def extract(_, carry):
        e = e_ref[...]
        m = jnp.min(e, axis=1, keepdims=True)
        pick = jnp.min(jnp.where(e == m, col, n_pad), axis=1, keepdims=True)
        e_ref[...] = jnp.where(col == pick, BIG, e)
        return carry

    lax.fori_loop(0, K_NBR, extract, 0)
    sel = (d2 <= R2) & jnp.logical_not(is_self) & (e_ref[...] >= BIG)
    sf = sel.astype(jnp.float32)
    indeg = jnp.sum(sf, axis=0, keepdims=True)
    sx = jnp.sum(sf * qx, axis=0, keepdims=True)
    sy = jnp.sum(sf * qy, axis=0, keepdims=True)

    @pl.when(i == 0)
    def _():
        out_ref[...] = jnp.zeros_like(out_ref)

    out_ref[0:1, :] = out_ref[0:1, :] + indeg
    out_ref[1:2, :] = out_ref[1:2, :] + sx
    out_ref[2:3, :] = out_ref[2:3, :] + sy


def _knn_stats(q, p_t, n_pad, row_base):
    br = KNN_BR
    nq = q.shape[0]
    return pl.pallas_call(
        functools.partial(_knn_body, n_pad=n_pad, br=br, row_base=row_base),
        grid=(nq // br,),
        in_specs=[
            pl.BlockSpec((br, 2), lambda i: (i, 0)),
            pl.BlockSpec((8, n_pad), lambda i: (0, 0)),
        ],
        out_specs=pl.BlockSpec((8, n_pad), lambda i: (0, 0)),
        out_shape=jax.ShapeDtypeStruct((8, n_pad), jnp.float32),
        scratch_shapes=[pltpu.VMEM((br, n_pad), jnp.float32)],
    )(q, p_t)


def _assemble_body(own_ref, pos_ref, sa_ref, st_ref, wa_ref, ca_ref,
                   wt_ref, ct_ref, inw_ref, inb_ref, h_ref, *, br):
    i = pl.program_id(0)
    gi = lax.broadcasted_iota(jnp.int32, (br, 1), 0) + i * br
    rowok = (gi < N_AG).astype(jnp.float32)
    pos = pos_ref[...]

    def six(s_ref, w_ref, c_ref):
        indeg = s_ref[:, 0:1]
        d = s_ref[:, 1:3] - pos * indeg
        return (jnp.dot(d, w_ref[...], preferred_element_type=jnp.float32)
                + indeg * c_ref[...])

    a6 = six(sa_ref, wa_ref, ca_ref)
    t6 = six(st_ref, wt_ref, ct_ref)
    state = jnp.concatenate([own_ref[...], a6, t6], axis=1) * rowok
    h_ref[...] = (jnp.dot(state, inw_ref[...],
                          preferred_element_type=jnp.float32) + inb_ref[...])


def _assemble(own_p, pos_p, sa, st, wa, ca, wt, ct, inw, inb):
    br = 512
    return pl.pallas_call(
        functools.partial(_assemble_body, br=br),
        grid=(NA_P // br,),
        in_specs=[
            pl.BlockSpec((br, 4), lambda i: (i, 0)),
            pl.BlockSpec((br, 2), lambda i: (i, 0)),
            pl.BlockSpec((br, 8), lambda i: (i, 0)),
            pl.BlockSpec((br, 8), lambda i: (i, 0)),
            pl.BlockSpec((2, 6), lambda i: (0, 0)),
            pl.BlockSpec((1, 6), lambda i: (0, 0)),
            pl.BlockSpec((2, 6), lambda i: (0, 0)),
            pl.BlockSpec((1, 6), lambda i: (0, 0)),
            pl.BlockSpec((16, NCH), lambda i: (0, 0)),
            pl.BlockSpec((1, NCH), lambda i: (0, 0)),
        ],
        out_specs=pl.BlockSpec((br, NCH), lambda i: (i, 0)),
        out_shape=jax.ShapeDtypeStruct((NA_P, NCH), jnp.float32),
    )(own_p, pos_p, sa, st, wa, ca, wt, ct, inw, inb)


_CHUNKS_PER_W = E_P // (NWORK * CH)
_ROWS_PER_TILE = NA_P // 16

def _edge_body(x_hbm, src_hbm, dst_hbm, w_hbm, out_hbm,
               src_v, dst_v, w_v, rows_v, acc_sh, sem):
    cid = lax.axis_index("c")
    sid = lax.axis_index("s")
    wid = sid * 2 + cid
    z16 = jnp.zeros((16,), jnp.float32)
    for r in range(CH):
        rows_v[r, pl.ds(0, 16)] = z16
        rows_v[r, pl.ds(16, 16)] = z16
    base_row = sid * _ROWS_PER_TILE
    for k in range(_ROWS_PER_TILE // CH):
        pltpu.sync_copy(rows_v, acc_sh.at[pl.ds(base_row + k * CH, CH)])
    plsc.subcore_barrier()

    def chunk(g, carry):
        base_e = wid * (_CHUNKS_PER_W * CH) + g * CH
        pltpu.sync_copy(src_hbm.at[pl.ds(base_e, CH)], src_v)
        pltpu.sync_copy(dst_hbm.at[pl.ds(base_e, CH)], dst_v)
        pltpu.sync_copy(w_hbm.at[pl.ds(base_e, CH)], w_v)
        pltpu.async_copy(x_hbm.at[src_v], rows_v, sem).wait()
        for r in range(CH):
            for s in (0, 16):
                rows_v[r, pl.ds(s, 16)] = (rows_v[r, pl.ds(s, 16)]
                                           * w_v[r, pl.ds(s, 16)])
        pltpu.sync_copy(rows_v, acc_sh.at[dst_v], add=True)
        return carry

    lax.fori_loop(0, _CHUNKS_PER_W, chunk, 0)
    plsc.subcore_barrier()
    for k in range(_ROWS_PER_TILE // CH):
        off = base_row + k * CH
        pltpu.sync_copy(acc_sh.at[pl.ds(off, CH)],
                        out_hbm.at[pl.ds(cid * NA_P + off, CH)])


_EDGE_KERNEL_CACHE = []


def _edge_pass(x, src, dst, w):
    if not _EDGE_KERNEL_CACHE:
        mesh = plsc.VectorSubcoreMesh(
            core_axis_name="c", subcore_axis_name="s",
            num_cores=2, num_subcores=16)
        _EDGE_KERNEL_CACHE.append(pl.kernel(
            _edge_body,
            out_type=jax.ShapeDtypeStruct((2 * NA_P, NCH), jnp.float32),
            mesh=mesh,
            compiler_params=pltpu.CompilerParams(use_tc_tiling_on_sc=False),
            scratch_types=[
                pltpu.VMEM((CH,), jnp.int32),
                pltpu.VMEM((CH,), jnp.int32),
                pltpu.VMEM((CH, NCH), jnp.float32),
                pltpu.VMEM((CH, NCH), jnp.float32),
                pltpu.VMEM_SHARED((NA_P, NCH), jnp.float32),
                pltpu.SemaphoreType.DMA,
            ],
        ))
    return _EDGE_KERNEL_CACHE[0](x, src, dst, w)


def _sum2_body(a_ref, b_ref, o_ref):
    o_ref[...] = a_ref[...] + b_ref[...]


def _sum2(parts):
    br = 1024
    nb = NA_P // br
    return pl.pallas_call(
        _sum2_body,
        grid=(nb,),
        in_specs=[
            pl.BlockSpec((br, NCH), lambda i: (i, 0)),
            pl.BlockSpec((br, NCH), lambda i: (i + nb, 0)),
        ],
        out_specs=pl.BlockSpec((br, NCH), lambda i: (i, 0)),
        out_shape=jax.ShapeDtypeStruct((NA_P, NCH), jnp.float32),
    )(parts, parts)


def _combine_body(*refs, br, readout):
    if readout:
        x0, x1, x2, x3, x4, w_ref, b_ref, ow_ref, ob_ref, o_ref = refs
    else:
        x0, x1, x2, x3, x4, w_ref, b_ref, o_ref = refs
    y = b_ref[...]
    for k, x in enumerate((x0, x1, x2, x3, x4)):
        y = y + jnp.dot(x[...], w_ref[k],
                        preferred_element_type=jnp.float32)
    h = jnp.where(y > 0, y, 0.01 * y)
    if readout:
        o_ref[...] = (jnp.dot(h, ow_ref[...],
                              preferred_element_type=jnp.float32)
                      + ob_ref[...])
    else:
        o_ref[...] = h


def _combine(xs, w, b, ow=None, ob=None):
    br = 512
    readout = ow is not None
    n_out = 2 if readout else NCH
    x_spec = pl.BlockSpec((br, NCH), lambda i: (i, 0))
    in_specs = [x_spec] * 5 + [
        pl.BlockSpec((5, NCH, NCH), lambda i: (0, 0, 0)),
        pl.BlockSpec((1, NCH), lambda i: (0, 0)),
    ]
    args = list(xs) + [w, b]
    if readout:
        in_specs += [
            pl.BlockSpec((NCH, 2), lambda i: (0, 0)),
            pl.BlockSpec((1, 2), lambda i: (0, 0)),
        ]
        args += [ow, ob]
    return pl.pallas_call(
        functools.partial(_combine_body, br=br, readout=readout),
        grid=(NA_P // br,),
        in_specs=in_specs,
        out_specs=pl.BlockSpec((br, n_out), lambda i: (i, 0)),
        out_shape=jax.ShapeDtypeStruct((NA_P, n_out), jnp.float32),
    )(*args)


def kernel(own_obs, agent_obs, target_obs, edge_index, edge_attr,
           phi_a_W, phi_a_b, rho_a_W, rho_a_b,
           phi_t_W, phi_t_b, rho_t_W, rho_t_b,
           gcn_in_W, gcn_in_b, gcn_taps_W, gcn_layer_b, gcn_out_W, gcn_out_b):
    f32 = jnp.float32
    q_a = jnp.concatenate(
        [agent_obs, jnp.full((NA_P - N_AG, 2), 1e6, f32)], axis=0)
    allp = jnp.concatenate([agent_obs, target_obs], axis=0)
    q_c = jnp.concatenate(
        [allp, jnp.full((NC_P - N_ALL, 2), 1e6, f32)], axis=0)
    q_t = jnp.concatenate(
        [target_obs, jnp.full((NT_P - N_TG, 2), 1e6, f32)], axis=0)
    pt_a = jnp.concatenate([q_a.T, jnp.zeros((6, NA_P), f32)], axis=0)
    pt_c = jnp.concatenate([q_c.T, jnp.zeros((6, NC_P), f32)], axis=0)

    stats_a = _knn_stats(q_a, pt_a, NA_P, 0)
    stats_c = _knn_stats(q_t, pt_c, NC_P, N_AG)
    sa = stats_a.T
    st = stats_c[:, :NA_P].T

    wa6 = phi_a_W @ rho_a_W
    ca6 = (phi_a_b @ rho_a_W + 0.0)[None, :]
    wt6 = phi_t_W @ rho_t_W
    ct6 = (phi_t_b @ rho_t_W)[None, :]
    inb_eff = (gcn_in_b + rho_a_b @ gcn_in_W[4:10]
               + rho_t_b @ gcn_in_W[10:16])[None, :]

    own_p = jnp.concatenate(
        [own_obs, jnp.zeros((NA_P - N_AG, 4), f32)], axis=0)
    h = _assemble(own_p, q_a, sa, st, wa6, ca6, wt6, ct6,
                  gcn_in_W, inb_eff)

    pad_i = jnp.zeros((E_P - E,), jnp.int32)
    src_p = jnp.concatenate([edge_index[0], pad_i])
    dst_p = jnp.concatenate([edge_index[1], pad_i])
    w_p = jnp.concatenate([edge_attr, jnp.zeros((E_P - E,), f32)])
    w_rows = jnp.broadcast_to(w_p[:, None], (E_P, NCH))

    for l in range(2):
        xs = [h]
        xk = h
        for _k in range(4):
            parts = _edge_pass(xk, src_p, dst_p, w_rows)
            xk = _sum2(parts)
            xs.append(xk)
        if l == 0:
            h = _combine(xs, gcn_taps_W[0], gcn_layer_b[0:1])
        else:
            out = _combine(xs, gcn_taps_W[1], gcn_layer_b[1:2],
                           gcn_out_W, gcn_out_b[None, :])
    return out[:N_AG]
